# baseline (speedup 1.0000x reference)
import functools

import jax
import jax.numpy as jnp
from jax import lax
from jax.experimental import pallas as pl
from jax.experimental.pallas import tpu as pltpu
from jax.experimental.pallas import tpu_sc as plsc

N_USER = 50000
N_ITEM = 50000
N_EDGES = 800000
FEATURE_DIM = 128
HIDDEN_DIM = 64
HH = 32

NC = 2
NS = 16
LANES = 16

CHUNK = 256
E_PAD = 819200
NCHUNKS = E_PAD // CHUNK
CPT = NCHUNKS // NS
SUP = 8
NSUP = CPT // SUP
NBUF = 3
DELAY = 1
ACC_ROWS = 50048
TRASH = 50000
ROWS_PER_TILE = ACC_ROWS // NS

_mesh = functools.partial(
    plsc.VectorSubcoreMesh, core_axis_name="c", subcore_axis_name="s",
    num_cores=NC, num_subcores=NS)


def _spmm_body(fa_hbm, fb_hbm, src_hbm, dst_hbm, zeros_hbm,
               outa_hbm, outb_hbm, idx_s, idx_d, rows, acc, sem_g, sem_s):
    c = lax.axis_index("c")
    t = lax.axis_index("s")
    r0 = t * ROWS_PER_TILE
    pltpu.sync_copy(zeros_hbm.at[pl.ds(r0, ROWS_PER_TILE)],
                    acc.at[pl.ds(r0, ROWS_PER_TILE)])
    plsc.subcore_barrier()

    chunk0 = t * CPT

    def super_body(si, carry):
        sc0 = chunk0 + si * SUP
        pltpu.sync_copy(src_hbm.at[pl.ds(sc0, SUP)], idx_s)
        pltpu.sync_copy(dst_hbm.at[pl.ds(sc0, SUP)], idx_d)
        def fire_gather(k, b):
            @pl.when(c == 0)
            def _():
                pltpu.async_copy(fa_hbm.at[idx_s.at[k]], rows.at[b], sem_g.at[b])

            @pl.when(c != 0)
            def _():
                pltpu.async_copy(fb_hbm.at[idx_s.at[k]], rows.at[b], sem_g.at[b])

        for k in range(NBUF):
            fire_gather(k, k)
        waited = [False] * SUP
        for k in range(SUP):
            b = k % NBUF
            pltpu.make_async_copy(fa_hbm.at[idx_s.at[k]], rows.at[b],
                                  sem_g.at[b]).wait()
            pltpu.async_copy(rows.at[b], acc.at[idx_d.at[k]], sem_s.at[b],
                             add=True)
            j = k - DELAY
            if j >= 0 and j + NBUF < SUP:
                bj = j % NBUF
                pltpu.make_async_copy(rows.at[bj], acc.at[idx_d.at[j]],
                                      sem_s.at[bj]).wait()
                waited[j] = True
                fire_gather(j + NBUF, bj)
        for k in range(SUP):
            if not waited[k]:
                b = k % NBUF
                pltpu.make_async_copy(rows.at[b], acc.at[idx_d.at[k]],
                                      sem_s.at[b]).wait()
        return carry

    lax.fori_loop(0, NSUP, super_body, 0)
    plsc.subcore_barrier()

    @pl.when(c == 0)
    def _():
        pltpu.sync_copy(acc.at[pl.ds(r0, ROWS_PER_TILE)],
                        outa_hbm.at[pl.ds(r0, ROWS_PER_TILE)])

    @pl.when(c != 0)
    def _():
        pltpu.sync_copy(acc.at[pl.ds(r0, ROWS_PER_TILE)],
                        outb_hbm.at[pl.ds(r0, ROWS_PER_TILE)])


def _spmm_sum(fa, fb, src2d, dst2d, zeros32):
    k = pl.kernel(
        _spmm_body,
        out_type=[jax.ShapeDtypeStruct((ACC_ROWS, HH), jnp.float32),
                  jax.ShapeDtypeStruct((ACC_ROWS, HH), jnp.float32)],
        mesh=_mesh(),
        compiler_params=pltpu.CompilerParams(use_tc_tiling_on_sc=False),
        scratch_types=[
            pltpu.VMEM((SUP, CHUNK), jnp.int32),
            pltpu.VMEM((SUP, CHUNK), jnp.int32),
            pltpu.VMEM((NBUF, CHUNK, HH), jnp.float32),
            pltpu.VMEM_SHARED((ACC_ROWS, HH), jnp.float32),
            pltpu.SemaphoreType.DMA((NBUF,)),
            pltpu.SemaphoreType.DMA((NBUF,)),
        ],
    )
    return k(fa, fb, src2d, dst2d, zeros32)


def _deg_body(uvd_hbm, vud_hbm, zeros_hbm, degu_hbm, degv_hbm,
              idx, ones, dacc, sem):
    c = lax.axis_index("c")
    t = lax.axis_index("s")
    r0 = t * ROWS_PER_TILE

    def fill_ones(i, carry):
        ones[i, :] = jnp.full((LANES,), 1.0, jnp.float32)
        return carry

    lax.fori_loop(0, CHUNK, fill_ones, 0)
    pltpu.sync_copy(zeros_hbm.at[pl.ds(r0, ROWS_PER_TILE)],
                    dacc.at[pl.ds(r0, ROWS_PER_TILE)])
    plsc.subcore_barrier()

    chunk0 = t * CPT

    def super_body(si, carry):
        sc0 = chunk0 + si * SUP

        @pl.when(c == 0)
        def _():
            pltpu.sync_copy(uvd_hbm.at[pl.ds(sc0, SUP)], idx)

        @pl.when(c != 0)
        def _():
            pltpu.sync_copy(vud_hbm.at[pl.ds(sc0, SUP)], idx)

        def chunk_body(k, carry2):
            pltpu.sync_copy(ones, dacc.at[idx.at[k]], add=True)
            return carry2

        return lax.fori_loop(0, SUP, chunk_body, carry)

    lax.fori_loop(0, NSUP, super_body, 0)
    plsc.subcore_barrier()

    @pl.when(c == 0)
    def _():
        pltpu.sync_copy(dacc.at[pl.ds(r0, ROWS_PER_TILE)],
                        degu_hbm.at[pl.ds(r0, ROWS_PER_TILE)])

    @pl.when(c != 0)
    def _():
        pltpu.sync_copy(dacc.at[pl.ds(r0, ROWS_PER_TILE)],
                        degv_hbm.at[pl.ds(r0, ROWS_PER_TILE)])


def _degrees(uv_dst2d, vu_dst2d, zeros16):
    k = pl.kernel(
        _deg_body,
        out_type=[jax.ShapeDtypeStruct((ACC_ROWS, LANES), jnp.float32),
                  jax.ShapeDtypeStruct((ACC_ROWS, LANES), jnp.float32)],
        mesh=_mesh(),
        compiler_params=pltpu.CompilerParams(use_tc_tiling_on_sc=False),
        scratch_types=[
            pltpu.VMEM((SUP, CHUNK), jnp.int32),
            pltpu.VMEM((CHUNK, LANES), jnp.float32),
            pltpu.VMEM_SHARED((ACC_ROWS, LANES), jnp.float32),
            pltpu.SemaphoreType.DMA,
        ],
    )
    return k(uv_dst2d, vu_dst2d, zeros16)


_BN = 1000
_NB = N_USER // _BN


def _embed_tc(x, W, b):
    def body(x_ref, w_ref, b_ref, oa_ref, ob_ref):
        y = jnp.dot(x_ref[...], w_ref[...],
                    preferred_element_type=jnp.float32) + b_ref[...]
        oa_ref[...] = y[:, :HH]
        ob_ref[...] = y[:, HH:]

    return pl.pallas_call(
        body,
        grid=(_NB,),
        in_specs=[
            pl.BlockSpec((_BN, FEATURE_DIM), lambda i: (i, 0)),
            pl.BlockSpec((FEATURE_DIM, HIDDEN_DIM), lambda i: (0, 0)),
            pl.BlockSpec((1, HIDDEN_DIM), lambda i: (0, 0)),
        ],
        out_specs=[
            pl.BlockSpec((_BN, HH), lambda i: (i, 0)),
            pl.BlockSpec((_BN, HH), lambda i: (i, 0)),
        ],
        out_shape=[jax.ShapeDtypeStruct((N_USER, HH), jnp.float32),
                   jax.ShapeDtypeStruct((N_USER, HH), jnp.float32)],
    )(x, W, b)


def _mid_tc(sa, sb, deg16, Wt, Wb, b, relu, split):
    def body(sa_ref, sb_ref, d_ref, wt_ref, wb_ref, b_ref, *outs):
        y = (jnp.dot(sa_ref[...], wt_ref[...], preferred_element_type=jnp.float32)
             + jnp.dot(sb_ref[...], wb_ref[...], preferred_element_type=jnp.float32))
        d = jnp.maximum(d_ref[...][:, :1], 1.0)
        y = y / d + b_ref[...]
        if relu:
            y = jnp.maximum(y, 0.0)
        if split:
            outs[0][...] = y[:, :HH]
            outs[1][...] = y[:, HH:]
        else:
            outs[0][...] = y

    if split:
        out_specs = [pl.BlockSpec((_BN, HH), lambda i: (i, 0)),
                     pl.BlockSpec((_BN, HH), lambda i: (i, 0))]
        out_shape = [jax.ShapeDtypeStruct((N_USER, HH), jnp.float32),
                     jax.ShapeDtypeStruct((N_USER, HH), jnp.float32)]
    else:
        out_specs = [pl.BlockSpec((_BN, HIDDEN_DIM), lambda i: (i, 0))]
        out_shape = [jax.ShapeDtypeStruct((N_USER, HIDDEN_DIM), jnp.float32)]

    res = pl.pallas_call(
        body,
        grid=(_NB,),
        in_specs=[
            pl.BlockSpec((_BN, HH), lambda i: (i, 0)),
            pl.BlockSpec((_BN, HH), lambda i: (i, 0)),
            pl.BlockSpec((_BN, LANES), lambda i: (i, 0)),
            pl.BlockSpec((HH, HIDDEN_DIM), lambda i: (0, 0)),
            pl.BlockSpec((HH, HIDDEN_DIM), lambda i: (0, 0)),
            pl.BlockSpec((1, HIDDEN_DIM), lambda i: (0, 0)),
        ],
        out_specs=out_specs,
        out_shape=out_shape,
    )(sa, sb, deg16, Wt, Wb, b)
    return res


def _pad_idx(idx, fill):
    idx = idx.astype(jnp.int32)
    pad = jnp.full((E_PAD - N_EDGES,), fill, jnp.int32)
    return jnp.concatenate([idx, pad]).reshape(NCHUNKS, CHUNK)


def kernel(ufea, vfea, UV_adj, VU_adj, adj, fake,
           W_user_embed, b_user_embed, W_item_embed, b_item_embed,
           Wu1, bu1, Wv1, bv1, Wu2, bu2, Wv2, bv2):
    del VU_adj, adj, fake
    uv_rows = UV_adj[0]
    uv_cols = UV_adj[1]

    uv_dst = _pad_idx(uv_rows, TRASH)
    uv_src = _pad_idx(uv_cols, 0)
    vu_dst = _pad_idx(uv_cols, TRASH)
    vu_src = _pad_idx(uv_rows, 0)

    zeros32 = jnp.zeros((ACC_ROWS, HH), jnp.float32)
    zeros16 = jnp.zeros((ACC_ROWS, LANES), jnp.float32)

    b_user = b_user_embed.reshape(1, HIDDEN_DIM)
    b_item = b_item_embed.reshape(1, HIDDEN_DIM)

    u0a, u0b = _embed_tc(ufea, W_user_embed, b_user)
    v0a, v0b = _embed_tc(vfea, W_item_embed, b_item)

    degu16, degv16 = _degrees(uv_dst, vu_dst, zeros16)

    su1a, su1b = _spmm_sum(v0a, v0b, uv_src, uv_dst, zeros32)
    sv1a, sv1b = _spmm_sum(u0a, u0b, vu_src, vu_dst, zeros32)

    u1a, u1b = _mid_tc(su1a, su1b, degu16, Wu1[:HH], Wu1[HH:],
                       bu1.reshape(1, HIDDEN_DIM), relu=True, split=True)
    v1a, v1b = _mid_tc(sv1a, sv1b, degv16, Wv1[:HH], Wv1[HH:],
                       bv1.reshape(1, HIDDEN_DIM), relu=True, split=True)

    su2a, su2b = _spmm_sum(v1a, v1b, uv_src, uv_dst, zeros32)
    sv2a, sv2b = _spmm_sum(u1a, u1b, vu_src, vu_dst, zeros32)

    (learn_user,) = _mid_tc(su2a, su2b, degu16, Wu2[:HH], Wu2[HH:],
                            bu2.reshape(1, HIDDEN_DIM), relu=False, split=False)
    (learn_item,) = _mid_tc(sv2a, sv2b, degv16, Wv2[:HH], Wv2[HH:],
                            bv2.reshape(1, HIDDEN_DIM), relu=False, split=False)
    return (learn_user, learn_item)


# speedup vs baseline: 1.5209x; 1.5209x over previous
import functools

import jax
import jax.numpy as jnp
from jax import lax
from jax.experimental import pallas as pl
from jax.experimental.pallas import tpu as pltpu
from jax.experimental.pallas import tpu_sc as plsc

N_USER = 50000
N_ITEM = 50000
N_EDGES = 800000
FEATURE_DIM = 128
HIDDEN_DIM = 64
HH = 32

NC = 2
NS = 16
LANES = 16

CHUNK = 256
E_PAD = 819200
NCHUNKS = E_PAD // CHUNK
CPT = NCHUNKS // NS
SUP = 8
NSUP = CPT // SUP
NBUF = 8
ACC_ROWS = 50048
TRASH = 50000
ROWS_PER_TILE = ACC_ROWS // NS

_mesh = functools.partial(
    plsc.VectorSubcoreMesh, core_axis_name="c", subcore_axis_name="s",
    num_cores=NC, num_subcores=NS)
_sc_params = pltpu.CompilerParams(use_tc_tiling_on_sc=False)


def _spmm_body(fa_hbm, fb_hbm, src_hbm, dst_hbm, zeros_hbm,
               outa_hbm, outb_hbm, idx_s, idx_d, rows, acc, sem_g, sem_s):
    c = lax.axis_index("c")
    t = lax.axis_index("s")
    r0 = t * ROWS_PER_TILE
    pltpu.sync_copy(zeros_hbm.at[pl.ds(r0, ROWS_PER_TILE)],
                    acc.at[pl.ds(r0, ROWS_PER_TILE)])
    plsc.subcore_barrier()

    chunk0 = t * CPT

    def fire_gather(k, b):
        @pl.when(c == 0)
        def _():
            pltpu.async_copy(fa_hbm.at[idx_s.at[k]], rows.at[b], sem_g.at[b])

        @pl.when(c != 0)
        def _():
            pltpu.async_copy(fb_hbm.at[idx_s.at[k]], rows.at[b], sem_g.at[b])

    def drain_scatters():
        for b in range(NBUF):
            pltpu.make_async_copy(rows.at[b], acc.at[idx_d.at[b]],
                                  sem_s.at[b]).wait()

    def super_body(si, carry):
        sc0 = chunk0 + si * SUP
        pltpu.sync_copy(src_hbm.at[pl.ds(sc0, SUP)], idx_s)
        pltpu.sync_copy(dst_hbm.at[pl.ds(sc0, SUP)], idx_d)

        @pl.when(si != 0)
        def _():
            drain_scatters()

        for k in range(SUP):
            fire_gather(k, k)
        for k in range(SUP):
            pltpu.make_async_copy(fa_hbm.at[idx_s.at[k]], rows.at[k],
                                  sem_g.at[k]).wait()
            pltpu.async_copy(rows.at[k], acc.at[idx_d.at[k]], sem_s.at[k],
                             add=True)
        return carry

    lax.fori_loop(0, NSUP, super_body, 0)
    drain_scatters()
    plsc.subcore_barrier()

    @pl.when(c == 0)
    def _():
        pltpu.sync_copy(acc.at[pl.ds(r0, ROWS_PER_TILE)],
                        outa_hbm.at[pl.ds(r0, ROWS_PER_TILE)])

    @pl.when(c != 0)
    def _():
        pltpu.sync_copy(acc.at[pl.ds(r0, ROWS_PER_TILE)],
                        outb_hbm.at[pl.ds(r0, ROWS_PER_TILE)])


def _spmm_sum(fa, fb, src2d, dst2d, zerosb):
    k = pl.kernel(
        _spmm_body,
        out_type=[jax.ShapeDtypeStruct((ACC_ROWS, HH), jnp.bfloat16),
                  jax.ShapeDtypeStruct((ACC_ROWS, HH), jnp.bfloat16)],
        mesh=_mesh(),
        compiler_params=_sc_params,
        scratch_types=[
            pltpu.VMEM((SUP, CHUNK), jnp.int32),
            pltpu.VMEM((SUP, CHUNK), jnp.int32),
            pltpu.VMEM((NBUF, CHUNK, HH), jnp.bfloat16),
            pltpu.VMEM_SHARED((ACC_ROWS, HH), jnp.bfloat16),
            pltpu.SemaphoreType.DMA((NBUF,)),
            pltpu.SemaphoreType.DMA((NBUF,)),
        ],
    )
    return k(fa, fb, src2d, dst2d, zerosb)


def _deg_body(uvd_hbm, vud_hbm, zeros_hbm, degu_hbm, degv_hbm,
              idx, ones, dacc, sem):
    c = lax.axis_index("c")
    t = lax.axis_index("s")
    r0 = t * ROWS_PER_TILE

    def fill_ones(i, carry):
        ones[i, :] = jnp.full((LANES,), 1.0, jnp.float32)
        return carry

    lax.fori_loop(0, CHUNK, fill_ones, 0)
    pltpu.sync_copy(zeros_hbm.at[pl.ds(r0, ROWS_PER_TILE)],
                    dacc.at[pl.ds(r0, ROWS_PER_TILE)])
    plsc.subcore_barrier()

    chunk0 = t * CPT

    def super_body(si, carry):
        sc0 = chunk0 + si * SUP

        @pl.when(c == 0)
        def _():
            pltpu.sync_copy(uvd_hbm.at[pl.ds(sc0, SUP)], idx)

        @pl.when(c != 0)
        def _():
            pltpu.sync_copy(vud_hbm.at[pl.ds(sc0, SUP)], idx)

        def chunk_body(k, carry2):
            pltpu.sync_copy(ones, dacc.at[idx.at[k]], add=True)
            return carry2

        return lax.fori_loop(0, SUP, chunk_body, carry)

    lax.fori_loop(0, NSUP, super_body, 0)
    plsc.subcore_barrier()

    @pl.when(c == 0)
    def _():
        pltpu.sync_copy(dacc.at[pl.ds(r0, ROWS_PER_TILE)],
                        degu_hbm.at[pl.ds(r0, ROWS_PER_TILE)])

    @pl.when(c != 0)
    def _():
        pltpu.sync_copy(dacc.at[pl.ds(r0, ROWS_PER_TILE)],
                        degv_hbm.at[pl.ds(r0, ROWS_PER_TILE)])


def _degrees(uv_dst2d, vu_dst2d, zeros16):
    k = pl.kernel(
        _deg_body,
        out_type=[jax.ShapeDtypeStruct((ACC_ROWS, LANES), jnp.float32),
                  jax.ShapeDtypeStruct((ACC_ROWS, LANES), jnp.float32)],
        mesh=_mesh(),
        compiler_params=_sc_params,
        scratch_types=[
            pltpu.VMEM((SUP, CHUNK), jnp.int32),
            pltpu.VMEM((CHUNK, LANES), jnp.float32),
            pltpu.VMEM_SHARED((ACC_ROWS, LANES), jnp.float32),
            pltpu.SemaphoreType.DMA,
        ],
    )
    return k(uv_dst2d, vu_dst2d, zeros16)


_BN = 1000
_NB = N_USER // _BN


def _embed_tc(x, W, b):
    def body(x_ref, w_ref, b_ref, oa_ref, ob_ref):
        y = jnp.dot(x_ref[...], w_ref[...],
                    preferred_element_type=jnp.float32) + b_ref[...]
        yb = y.astype(jnp.bfloat16)
        oa_ref[...] = yb[:, :HH]
        ob_ref[...] = yb[:, HH:]

    return pl.pallas_call(
        body,
        grid=(_NB,),
        in_specs=[
            pl.BlockSpec((_BN, FEATURE_DIM), lambda i: (i, 0)),
            pl.BlockSpec((FEATURE_DIM, HIDDEN_DIM), lambda i: (0, 0)),
            pl.BlockSpec((1, HIDDEN_DIM), lambda i: (0, 0)),
        ],
        out_specs=[
            pl.BlockSpec((_BN, HH), lambda i: (i, 0)),
            pl.BlockSpec((_BN, HH), lambda i: (i, 0)),
        ],
        out_shape=[jax.ShapeDtypeStruct((N_USER, HH), jnp.bfloat16),
                   jax.ShapeDtypeStruct((N_USER, HH), jnp.bfloat16)],
    )(x, W, b)


def _mid_tc(sa, sb, deg16, Wt, Wb, b, relu, split):
    def body(sa_ref, sb_ref, d_ref, wt_ref, wb_ref, b_ref, *outs):
        sa32 = sa_ref[...].astype(jnp.float32)
        sb32 = sb_ref[...].astype(jnp.float32)
        y = (jnp.dot(sa32, wt_ref[...], preferred_element_type=jnp.float32)
             + jnp.dot(sb32, wb_ref[...], preferred_element_type=jnp.float32))
        d = jnp.maximum(d_ref[...][:, :1], 1.0)
        y = y / d + b_ref[...]
        if relu:
            y = jnp.maximum(y, 0.0)
        if split:
            yb = y.astype(jnp.bfloat16)
            outs[0][...] = yb[:, :HH]
            outs[1][...] = yb[:, HH:]
        else:
            outs[0][...] = y

    if split:
        out_specs = [pl.BlockSpec((_BN, HH), lambda i: (i, 0)),
                     pl.BlockSpec((_BN, HH), lambda i: (i, 0))]
        out_shape = [jax.ShapeDtypeStruct((N_USER, HH), jnp.bfloat16),
                     jax.ShapeDtypeStruct((N_USER, HH), jnp.bfloat16)]
    else:
        out_specs = [pl.BlockSpec((_BN, HIDDEN_DIM), lambda i: (i, 0))]
        out_shape = [jax.ShapeDtypeStruct((N_USER, HIDDEN_DIM), jnp.float32)]

    return pl.pallas_call(
        body,
        grid=(_NB,),
        in_specs=[
            pl.BlockSpec((_BN, HH), lambda i: (i, 0)),
            pl.BlockSpec((_BN, HH), lambda i: (i, 0)),
            pl.BlockSpec((_BN, LANES), lambda i: (i, 0)),
            pl.BlockSpec((HH, HIDDEN_DIM), lambda i: (0, 0)),
            pl.BlockSpec((HH, HIDDEN_DIM), lambda i: (0, 0)),
            pl.BlockSpec((1, HIDDEN_DIM), lambda i: (0, 0)),
        ],
        out_specs=out_specs,
        out_shape=out_shape,
    )(sa, sb, deg16, Wt, Wb, b)


def _pad_idx(idx, fill):
    idx = idx.astype(jnp.int32)
    pad = jnp.full((E_PAD - N_EDGES,), fill, jnp.int32)
    return jnp.concatenate([idx, pad]).reshape(NCHUNKS, CHUNK)


def kernel(ufea, vfea, UV_adj, VU_adj, adj, fake,
           W_user_embed, b_user_embed, W_item_embed, b_item_embed,
           Wu1, bu1, Wv1, bv1, Wu2, bu2, Wv2, bv2):
    del VU_adj, adj, fake
    uv_rows = UV_adj[0]
    uv_cols = UV_adj[1]

    uv_dst = _pad_idx(uv_rows, TRASH)
    uv_src = _pad_idx(uv_cols, 0)
    vu_dst = _pad_idx(uv_cols, TRASH)
    vu_src = _pad_idx(uv_rows, 0)

    zerosb = jnp.zeros((ACC_ROWS, HH), jnp.bfloat16)
    zeros16 = jnp.zeros((ACC_ROWS, LANES), jnp.float32)

    b_user = b_user_embed.reshape(1, HIDDEN_DIM)
    b_item = b_item_embed.reshape(1, HIDDEN_DIM)

    u0a, u0b = _embed_tc(ufea, W_user_embed, b_user)
    v0a, v0b = _embed_tc(vfea, W_item_embed, b_item)

    degu16, degv16 = _degrees(uv_dst, vu_dst, zeros16)

    su1a, su1b = _spmm_sum(v0a, v0b, uv_src, uv_dst, zerosb)
    sv1a, sv1b = _spmm_sum(u0a, u0b, vu_src, vu_dst, zerosb)

    u1a, u1b = _mid_tc(su1a, su1b, degu16, Wu1[:HH], Wu1[HH:],
                       bu1.reshape(1, HIDDEN_DIM), relu=True, split=True)
    v1a, v1b = _mid_tc(sv1a, sv1b, degv16, Wv1[:HH], Wv1[HH:],
                       bv1.reshape(1, HIDDEN_DIM), relu=True, split=True)

    su2a, su2b = _spmm_sum(v1a, v1b, uv_src, uv_dst, zerosb)
    sv2a, sv2b = _spmm_sum(u1a, u1b, vu_src, vu_dst, zerosb)

    (learn_user,) = _mid_tc(su2a, su2b, degu16, Wu2[:HH], Wu2[HH:],
                            bu2.reshape(1, HIDDEN_DIM), relu=False, split=False)
    (learn_item,) = _mid_tc(sv2a, sv2b, degv16, Wv2[:HH], Wv2[HH:],
                            bv2.reshape(1, HIDDEN_DIM), relu=False, split=False)
    return (learn_user, learn_item)
